# Initial kernel scaffold; baseline (speedup 1.0000x reference)
import jax
import jax.numpy as jnp
from jax.experimental import pallas as pl


def kernel(features_s, features_t, labels_s, edge_index_s, edge_index_t, W0, b0, W1, b1, Wf, bf, Wd, bd):
    raise NotImplementedError("write your pallas kernel here")


# speedup vs baseline: 13.6160x; 13.6160x over previous
import functools

import jax
import jax.numpy as jnp
from jax import lax
from jax.experimental import pallas as pl
from jax.experimental.pallas import tpu as pltpu
from jax.experimental.pallas import tpu_sc as plsc

N = 10000
NP = 10240
E = 320000
D = 128
NC = 2
NS = 16
EPT = E // NS
K = 80
NB = EPT // K
RPT = NP // NS

BR = 2048
GRID = (2 * NP) // BR

_mesh = plsc.VectorSubcoreMesh(
    core_axis_name="c", subcore_axis_name="s", num_cores=NC, num_subcores=NS)


def _deg_body(dst_s, dst_t, deg_out, acc_v, idx_v, red_v, out_v, shared):
    sid = lax.axis_index("s")
    cid = lax.axis_index("c")
    zero16 = jnp.zeros((16,), jnp.float32)
    ones16 = jnp.ones((16,), jnp.float32)

    def zero_acc(i, _):
        acc_v[pl.ds(i * 16, 16)] = zero16
        return 0
    lax.fori_loop(0, NP // 16, zero_acc, 0)

    def process(dst):
        base = sid * EPT

        def chunk(ci, _):
            pltpu.sync_copy(dst.at[pl.ds(base + ci * 2000, 2000)], idx_v)

            def inner(j, _):
                idx = idx_v[pl.ds(j * 16, 16)]
                plsc.addupdate_scatter(acc_v, [idx], ones16)
                return 0
            lax.fori_loop(0, 125, inner, 0)
            return 0
        lax.fori_loop(0, EPT // 2000, chunk, 0)

    @pl.when(cid == 0)
    def _():
        process(dst_s)

    @pl.when(cid == 1)
    def _():
        process(dst_t)

    pltpu.sync_copy(acc_v, shared.at[sid])
    plsc.subcore_barrier()
    pltpu.sync_copy(shared.at[:, pl.ds(sid * RPT, RPT)], red_v)

    def red_col(cc, _):
        s = red_v[0, pl.ds(cc * 16, 16)]
        for r in range(1, NS):
            s = s + red_v[r, pl.ds(cc * 16, 16)]
        out_v[pl.ds(cc * 16, 16)] = s
        return 0
    lax.fori_loop(0, RPT // 16, red_col, 0)

    @pl.when(cid == 0)
    def _():
        pltpu.sync_copy(out_v, deg_out.at[0, pl.ds(sid * RPT, RPT)])

    @pl.when(cid == 1)
    def _():
        pltpu.sync_copy(out_v, deg_out.at[1, pl.ds(sid * RPT, RPT)])


_deg_call = pl.kernel(
    _deg_body,
    out_type=jax.ShapeDtypeStruct((NC, NP), jnp.float32),
    mesh=_mesh,
    scratch_types=[
        pltpu.VMEM((NP,), jnp.float32),
        pltpu.VMEM((2000,), jnp.int32),
        pltpu.VMEM((NS, RPT), jnp.float32),
        pltpu.VMEM((RPT,), jnp.float32),
        pltpu.VMEM_SHARED((NS, NP), jnp.float32),
    ],
    compiler_params=pltpu.CompilerParams(needs_layout_passes=False),
)


CH = 2000


def _mp_body(hp, src_s, dst_s, src_t, dst_t, agg, srcc, dstc, src_v, dst_v,
             rows_v, acc, sem):
    sid = lax.axis_index("s")
    cid = lax.axis_index("c")
    zero16 = jnp.zeros((16,), jnp.float32)

    def zero_rows(i, _):
        rows_v[i, pl.ds(0, 16)] = zero16
        for j in range(1, D // 16):
            rows_v[i, pl.ds(j * 16, 16)] = zero16
        return 0
    lax.fori_loop(0, K, zero_rows, 0)
    for k in range(RPT // K):
        pltpu.sync_copy(rows_v, acc.at[pl.ds(sid * RPT + k * K, K)])
    plsc.subcore_barrier()

    def process(src, dst, off):
        base = sid * EPT

        def chunk(ci, _):
            pltpu.sync_copy(src.at[pl.ds(base + ci * CH, CH)], srcc)
            pltpu.sync_copy(dst.at[pl.ds(base + ci * CH, CH)], dstc)

            def batch(bi, _):
                e0 = bi * K
                for j in range(K // 16):
                    src_v[pl.ds(j * 16, 16)] = srcc[pl.ds(e0 + j * 16, 16)] + off
                    dst_v[pl.ds(j * 16, 16)] = dstc[pl.ds(e0 + j * 16, 16)]
                pltpu.async_copy(hp.at[src_v], rows_v, sem).wait()
                pltpu.sync_copy(rows_v, acc.at[dst_v], add=True)
                return 0
            lax.fori_loop(0, CH // K, batch, 0)
            return 0
        lax.fori_loop(0, EPT // CH, chunk, 0)

    @pl.when(cid == 0)
    def _():
        process(src_s, dst_s, 0)

    @pl.when(cid == 1)
    def _():
        process(src_t, dst_t, NP)

    plsc.subcore_barrier()

    @pl.when(cid == 0)
    def _():
        pltpu.sync_copy(acc.at[pl.ds(sid * RPT, RPT)],
                        agg.at[0, pl.ds(sid * RPT, RPT)])

    @pl.when(cid == 1)
    def _():
        pltpu.sync_copy(acc.at[pl.ds(sid * RPT, RPT)],
                        agg.at[1, pl.ds(sid * RPT, RPT)])


_mp_call = pl.kernel(
    _mp_body,
    out_type=jax.ShapeDtypeStruct((NC, NP, D), jnp.float32),
    mesh=_mesh,
    scratch_types=[
        pltpu.VMEM((CH,), jnp.int32),
        pltpu.VMEM((CH,), jnp.int32),
        pltpu.VMEM((K,), jnp.int32),
        pltpu.VMEM((K,), jnp.int32),
        pltpu.VMEM((K, D), jnp.float32),
        pltpu.VMEM_SHARED((NP, D), jnp.float32),
        pltpu.SemaphoreType.DMA,
    ],
)


def _isd(deg_blk):
    return lax.rsqrt(jnp.maximum(deg_blk, 1.0))


def _tc1_body(f_ref, w_ref, deg_ref, hp_ref):
    isd = _isd(deg_ref[...])
    hp_ref[...] = jnp.dot(f_ref[...], w_ref[...],
                          preferred_element_type=jnp.float32) * isd


_tc1_call = pl.pallas_call(
    _tc1_body,
    grid=(GRID,),
    in_specs=[
        pl.BlockSpec((BR, D), lambda i: (i, 0)),
        pl.BlockSpec((D, D), lambda i: (0, 0)),
        pl.BlockSpec((BR, 1), lambda i: (i, 0)),
    ],
    out_specs=pl.BlockSpec((BR, D), lambda i: (i, 0)),
    out_shape=jax.ShapeDtypeStruct((2 * NP, D), jnp.float32),
)


def _tc2_body(agg_ref, deg_ref, b_ref, w_ref, x1_ref, hp_ref):
    isd = _isd(deg_ref[...])
    x = agg_ref[...] * isd + b_ref[...]
    x1 = jnp.where(x >= 0, x, 0.01 * x)
    x1_ref[...] = x1
    hp_ref[...] = jnp.dot(x1, w_ref[...],
                          preferred_element_type=jnp.float32) * isd


_tc2_call = pl.pallas_call(
    _tc2_body,
    grid=(GRID,),
    in_specs=[
        pl.BlockSpec((BR, D), lambda i: (i, 0)),
        pl.BlockSpec((BR, 1), lambda i: (i, 0)),
        pl.BlockSpec((1, D), lambda i: (0, 0)),
        pl.BlockSpec((D, D), lambda i: (0, 0)),
    ],
    out_specs=[
        pl.BlockSpec((BR, D), lambda i: (i, 0)),
        pl.BlockSpec((BR, D), lambda i: (i, 0)),
    ],
    out_shape=[
        jax.ShapeDtypeStruct((2 * NP, D), jnp.float32),
        jax.ShapeDtypeStruct((2 * NP, D), jnp.float32),
    ],
)


def _tc3_body(agg_ref, deg_ref, b_ref, x1_ref, wf_ref, bf_ref, wdt_ref,
              lab_ref, out_ref, acc_sm):
    pi = pl.program_id(0)

    @pl.when(pi == 0)
    def _():
        acc_sm[0] = 0.0
        acc_sm[1] = 0.0

    isd = _isd(deg_ref[...])
    x = agg_ref[...] * isd + b_ref[...]
    x2 = jnp.where(x >= 0, x, 0.01 * x)
    logits = jnp.dot(x2, wf_ref[...],
                     preferred_element_type=jnp.float32) + bf_ref[...]

    col = lax.broadcasted_iota(jnp.int32, (BR, D), 1)
    is_cls = col < 10
    lg_m = jnp.where(is_cls, logits, jnp.float32(-1e30))
    m = jnp.max(lg_m, axis=1, keepdims=True)
    ex = jnp.where(is_cls, jnp.exp(logits - m), 0.0)
    lse = m + jnp.log(jnp.sum(ex, axis=1, keepdims=True))

    row = lax.broadcasted_iota(jnp.int32, (BR, 1), 0) + pi * BR
    is_s = row < N
    lab = lab_ref[...]
    sel = jnp.sum(jnp.where(col == lab, logits, 0.0), axis=1, keepdims=True)
    cls_part = jnp.sum(jnp.where(is_s, sel - lse, 0.0))

    x1 = x1_ref[...]
    wdt = wdt_ref[...]

    def dcol(a):
        z = (jnp.sum(x1 * wdt[a:a + 1, :], axis=1, keepdims=True)
             + jnp.sum(x2 * wdt[a + 2:a + 3, :], axis=1, keepdims=True)
             + jnp.sum(logits * wdt[a + 4:a + 5, :], axis=1, keepdims=True)
             + jnp.sum(wdt[a + 6:a + 7, :]))
        return z

    z0 = dcol(0)
    z1 = dcol(1)
    mz = jnp.maximum(z0, z1)
    lse2 = mz + jnp.log(jnp.exp(z0 - mz) + jnp.exp(z1 - mz))
    zsel = jnp.where(is_s, z0, z1)
    valid = is_s | ((row >= NP) & (row < NP + N))
    dom_part = jnp.sum(jnp.where(valid, zsel - lse2, 0.0))

    acc_sm[0] = acc_sm[0] + cls_part
    acc_sm[1] = acc_sm[1] + dom_part

    @pl.when(pi == GRID - 1)
    def _():
        loss = (-(acc_sm[0] / jnp.float32(N))
                - 0.02 * (acc_sm[1] / jnp.float32(2 * N)))
        out_ref[...] = jnp.full((1, 1), loss, jnp.float32)


_tc3_call = pl.pallas_call(
    _tc3_body,
    grid=(GRID,),
    in_specs=[
        pl.BlockSpec((BR, D), lambda i: (i, 0)),
        pl.BlockSpec((BR, 1), lambda i: (i, 0)),
        pl.BlockSpec((1, D), lambda i: (0, 0)),
        pl.BlockSpec((BR, D), lambda i: (i, 0)),
        pl.BlockSpec((D, D), lambda i: (0, 0)),
        pl.BlockSpec((1, D), lambda i: (0, 0)),
        pl.BlockSpec((8, D), lambda i: (0, 0)),
        pl.BlockSpec((BR, 1), lambda i: (i, 0)),
    ],
    out_specs=pl.BlockSpec((1, 1), lambda i: (0, 0)),
    out_shape=jax.ShapeDtypeStruct((1, 1), jnp.float32),
    scratch_shapes=[pltpu.SMEM((2,), jnp.float32)],
)


def kernel(features_s, features_t, labels_s, edge_index_s, edge_index_t,
           W0, b0, W1, b1, Wf, bf, Wd, bd):
    f32 = jnp.float32
    ei_s = edge_index_s.astype(jnp.int32)
    ei_t = edge_index_t.astype(jnp.int32)

    F = (jnp.zeros((2 * NP, D), f32)
         .at[0:N].set(features_s.astype(f32))
         .at[NP:NP + N].set(features_t.astype(f32)))
    lab_col = (jnp.zeros((2 * NP, 1), jnp.int32)
               .at[0:N, 0].set(labels_s.astype(jnp.int32)))
    wf_pad = jnp.zeros((D, D), f32).at[:, :10].set(Wf)
    bf_row = jnp.zeros((1, D), f32).at[0, :10].set(bf)
    wdt = (jnp.zeros((8, D), f32)
           .at[0, :].set(Wd[:D, 0]).at[1, :].set(Wd[:D, 1])
           .at[2, :].set(Wd[D:2 * D, 0]).at[3, :].set(Wd[D:2 * D, 1])
           .at[4, :10].set(Wd[2 * D:, 0]).at[5, :10].set(Wd[2 * D:, 1])
           .at[6, 0].set(bd[0]).at[7, 0].set(bd[1]))
    b0_row = b0.reshape(1, D)
    b1_row = b1.reshape(1, D)

    src_s, dst_s = ei_s[0], ei_s[1]
    src_t, dst_t = ei_t[0], ei_t[1]

    deg2 = _deg_call(dst_s, dst_t)
    deg_col = deg2.reshape(2 * NP, 1)

    hp0 = _tc1_call(F, W0, deg_col)
    agg0 = _mp_call(hp0, src_s, dst_s, src_t, dst_t).reshape(2 * NP, D)
    x1, hp1 = _tc2_call(agg0, deg_col, b0_row, W1)
    agg1 = _mp_call(hp1, src_s, dst_s, src_t, dst_t).reshape(2 * NP, D)
    out = _tc3_call(agg1, deg_col, b1_row, x1, wf_pad, bf_row, wdt, lab_col)
    return out.reshape(())
